# baseline (speedup 1.0000x reference)
import functools

import jax
import jax.numpy as jnp
from jax import lax
from jax.experimental import pallas as pl
from jax.experimental.pallas import tpu as pltpu
from jax.experimental.pallas import tpu_sc as plsc

NUM_CORES = 2
NUM_SUBCORES = 16
NW = NUM_CORES * NUM_SUBCORES
LANES = 16
CHUNK = 128
TILE_SM = 8
TILE_MIN = 128


def _sc_gmf(uidx_hbm, iidx_hbm, utab_hbm, itab_hbm, w_hbm, b_hbm, out_hbm,
            uidx_v, iidx_v, u_fl, i_fl, w_v, b_v, out_v,
            sem, *, bpw, d, v):
  n_chunks = bpw // CHUNK
  wid = lax.axis_index("s") * NUM_CORES + lax.axis_index("c")
  base = wid * bpw

  pltpu.sync_copy(uidx_hbm.at[pl.ds(base, bpw)], uidx_v)
  pltpu.sync_copy(iidx_hbm.at[pl.ds(base, bpw)], iidx_v)
  pltpu.sync_copy(w_hbm, w_v)
  pltpu.sync_copy(b_hbm, b_v)

  def streams(dd, fn):
    for j in range(n_chunks):
      cs = pl.ds(j * CHUNK, CHUNK)
      ds_ = pl.ds(dd * bpw + j * CHUNK, CHUNK)
      fn(utab_hbm.at[dd].at[uidx_v.at[cs]], u_fl.at[ds_])
      fn(itab_hbm.at[dd].at[iidx_v.at[cs]], i_fl.at[ds_])

  def issue(dd):
    streams(dd, lambda s_, d_: pltpu.async_copy(s_, d_, sem))

  def drain(dd):
    streams(dd, lambda s_, d_: pltpu.make_async_copy(s_, d_, sem).wait())

  def dma_body(dd, carry):
    issue(dd)
    drain(dd - 1)
    return carry

  issue(0)
  lax.fori_loop(1, d, dma_body, 0, unroll=False)
  drain(d - 1)

  b_vec = b_v[...]
  w_chunks = [w_v[pl.ds(k * LANES, LANES)] for k in range(d // LANES)]
  wb = [jnp.broadcast_to(w_chunks[dd // LANES][dd % LANES], (LANES,))
        for dd in range(d)]
  lanes = lax.iota(jnp.int32, LANES)
  zeros = jnp.zeros((LANES,), jnp.int32)

  def body(t, carry):
    pos0 = t * LANES + lanes
    acc = jnp.zeros((LANES,), jnp.float32)
    for dd in range(d):
      pos = pos0 + dd * bpw
      uv = plsc.load_gather(u_fl, [pos])
      iv = plsc.load_gather(i_fl, [pos])
      acc = acc + (uv * iv) * wb[dd]
    logit = acc + b_vec
    out_v[pl.ds(t * LANES, LANES)] = 1.0 / (1.0 + jnp.exp(-logit))
    return carry

  lax.fori_loop(0, bpw // LANES, body, 0)
  pltpu.sync_copy(out_v, out_hbm.at[pl.ds(base, bpw)])


def kernel(user_indices, item_indices, user_table, item_table, W, b):
  B = user_indices.shape[0]
  V, D = user_table.shape
  bpw = B // NW

  uidx = user_indices.astype(jnp.int32)
  iidx = item_indices.astype(jnp.int32)
  ut = user_table.T
  it = item_table.T
  w_flat = W.reshape(D).astype(jnp.float32)
  b_vec = jnp.broadcast_to(b.astype(jnp.float32), (LANES,))

  mesh = plsc.VectorSubcoreMesh(core_axis_name="c", subcore_axis_name="s")
  sc = functools.partial(
      pl.kernel,
      mesh=mesh,
      compiler_params=pltpu.CompilerParams(
          needs_layout_passes=False, use_tc_tiling_on_sc=False),
      out_type=jax.ShapeDtypeStruct((B,), jnp.float32),
      scratch_types=[
          pltpu.VMEM((bpw,), jnp.int32),
          pltpu.VMEM((bpw,), jnp.int32),
          pltpu.VMEM((D * bpw,), jnp.float32),
          pltpu.VMEM((D * bpw,), jnp.float32),
          pltpu.VMEM((D,), jnp.float32),
          pltpu.VMEM((LANES,), jnp.float32),
          pltpu.VMEM((bpw,), jnp.float32),
          pltpu.SemaphoreType.DMA,
      ],
  )(functools.partial(_sc_gmf, bpw=bpw, d=D, v=V))

  out = sc(uidx, iidx, ut, it, w_flat, b_vec)
  return out.reshape(B, 1)


# speedup vs baseline: 23.0676x; 23.0676x over previous
import functools

import jax
import jax.numpy as jnp
from jax import lax
from jax.experimental import pallas as pl
from jax.experimental.pallas import tpu as pltpu
from jax.experimental.pallas import tpu_sc as plsc

NUM_CORES = 2
NUM_SUBCORES = 16
NW = NUM_CORES * NUM_SUBCORES
LANES = 16
RING = 8
GRP = 16


def _sc_gmf(uidx_hbm, iidx_hbm, utab_hbm, itab_hbm, w_hbm, b_hbm, out_hbm,
            uidx_v, iidx_v, dlist_v, du_v, di_v, w_v, b_v, out_v,
            *sems, bpw, d):
  wid = lax.axis_index("s") * NUM_CORES + lax.axis_index("c")
  base = wid * bpw

  pltpu.sync_copy(uidx_hbm.at[pl.ds(base, bpw)], uidx_v)
  pltpu.sync_copy(iidx_hbm.at[pl.ds(base, bpw)], iidx_v)
  pltpu.sync_copy(w_hbm, w_v)
  pltpu.sync_copy(b_hbm, b_v)

  lanes = lax.iota(jnp.int32, LANES)
  dlist_v[pl.ds(0, LANES)] = lanes
  dlist_v[pl.ds(LANES, LANES)] = lanes + LANES

  w_lo = w_v[pl.ds(0, LANES)]
  w_hi = w_v[pl.ds(LANES, LANES)]
  lane0 = lanes == 0

  def fill_and_issue(uvec, ivec, k, slot):
    r_u = uvec[k]
    r_i = ivec[k]
    cu = pl.multiple_of(
        lax.shift_left(lax.shift_right_logical(r_u, 7), 7), 128)
    ci = pl.multiple_of(
        lax.shift_left(lax.shift_right_logical(r_i, 7), 7), 128)
    pltpu.async_copy(utab_hbm.at[dlist_v.at[:], pl.ds(cu, 128)],
                     du_v.at[pl.ds(slot * 32, 32)], sems[slot])
    pltpu.async_copy(itab_hbm.at[dlist_v.at[:], pl.ds(ci, 128)],
                     di_v.at[pl.ds(slot * 32, 32)], sems[slot])

  def wait_and_consume(uvec, ivec, k, slot, e):
    pltpu.make_async_copy(utab_hbm.at[dlist_v.at[:], pl.ds(0, 128)],
                          du_v.at[pl.ds(slot * 32, 32)], sems[slot]).wait()
    pltpu.make_async_copy(itab_hbm.at[dlist_v.at[:], pl.ds(0, 128)],
                          di_v.at[pl.ds(slot * 32, 32)], sems[slot]).wait()
    ucol = jnp.broadcast_to(jnp.bitwise_and(uvec[k], 127), (LANES,))
    icol = jnp.broadcast_to(jnp.bitwise_and(ivec[k], 127), (LANES,))
    ulo = plsc.load_gather(du_v, [lanes + slot * 32, ucol])
    uhi = plsc.load_gather(du_v, [lanes + slot * 32 + LANES, ucol])
    ilo = plsc.load_gather(di_v, [lanes + slot * 32, icol])
    ihi = plsc.load_gather(di_v, [lanes + slot * 32 + LANES, icol])
    s = jnp.sum((ulo * ilo) * w_lo + (uhi * ihi) * w_hi)
    plsc.store_scatter(out_v, [jnp.full((LANES,), e, jnp.int32)],
                       jnp.broadcast_to(s, (LANES,)), mask=lane0)

  n_groups = bpw // GRP

  uvec0 = uidx_v[pl.ds(0, GRP)]
  ivec0 = iidx_v[pl.ds(0, GRP)]
  for k in range(RING):
    fill_and_issue(uvec0, ivec0, k, k)

  def body(g, carry):
    uvec = uidx_v[pl.ds(g * GRP, GRP)]
    ivec = iidx_v[pl.ds(g * GRP, GRP)]
    gm1 = g - 1
    uvp = uidx_v[pl.ds(gm1 * GRP, GRP)]
    ivp = iidx_v[pl.ds(gm1 * GRP, GRP)]
    for k in range(RING):
      wait_and_consume(uvp, ivp, RING + k, k, gm1 * GRP + RING + k)
      fill_and_issue(uvec, ivec, k, k)
    for k in range(RING):
      wait_and_consume(uvec, ivec, k, k, g * GRP + k)
      fill_and_issue(uvec, ivec, RING + k, k)
    return carry

  for k in range(RING):
    wait_and_consume(uvec0, ivec0, k, k, k)
    fill_and_issue(uvec0, ivec0, RING + k, k)

  lax.fori_loop(1, n_groups, body, 0, unroll=False)

  glast = n_groups - 1
  uvl = uidx_v[pl.ds(glast * GRP, GRP)]
  ivl = iidx_v[pl.ds(glast * GRP, GRP)]
  for k in range(RING):
    wait_and_consume(uvl, ivl, RING + k, k, glast * GRP + RING + k)

  b_vec = b_v[...]

  def sig_body(t, carry):
    s_ = pl.ds(t * LANES, LANES)
    logit = out_v[s_] + b_vec
    out_v[s_] = 1.0 / (1.0 + jnp.exp(-logit))
    return carry

  lax.fori_loop(0, bpw // LANES, sig_body, 0)
  pltpu.sync_copy(out_v, out_hbm.at[pl.ds(base, bpw)])


def kernel(user_indices, item_indices, user_table, item_table, W, b):
  B = user_indices.shape[0]
  V, D = user_table.shape
  bpw = B // NW

  uidx = user_indices.astype(jnp.int32)
  iidx = item_indices.astype(jnp.int32)
  ut = user_table.T
  it = item_table.T
  w_flat = W.reshape(D).astype(jnp.float32)
  b_vec = jnp.broadcast_to(b.astype(jnp.float32), (LANES,))

  mesh = plsc.VectorSubcoreMesh(core_axis_name="c", subcore_axis_name="s")
  sc = functools.partial(
      pl.kernel,
      mesh=mesh,
      compiler_params=pltpu.CompilerParams(
          needs_layout_passes=False, use_tc_tiling_on_sc=True),
      out_type=jax.ShapeDtypeStruct((B,), jnp.float32),
      scratch_types=[
          pltpu.VMEM((bpw,), jnp.int32),
          pltpu.VMEM((bpw,), jnp.int32),
          pltpu.VMEM((D,), jnp.int32),
          pltpu.VMEM((RING * 32, 128), jnp.float32),
          pltpu.VMEM((RING * 32, 128), jnp.float32),
          pltpu.VMEM((D,), jnp.float32),
          pltpu.VMEM((LANES,), jnp.float32),
          pltpu.VMEM((bpw,), jnp.float32),
      ] + [pltpu.SemaphoreType.DMA] * RING,
  )(functools.partial(_sc_gmf, bpw=bpw, d=D))

  out = sc(uidx, iidx, ut, it, w_flat, b_vec)
  return out.reshape(B, 1)
